# Initial kernel scaffold; baseline (speedup 1.0000x reference)
import functools

import jax
import jax.numpy as jnp
from jax import lax
from jax.experimental import pallas as pl
from jax.experimental.pallas import tpu as pltpu
from jax.experimental.pallas import tpu_sc as plsc

VOCAB = 1000000
HIDDEN = 64
L = 200
B = 1024
NW = 32
ROWS_PER_W = B // NW
CHUNK = L
LANES = 16
VPR = HIDDEN // LANES


def _body(ids_hbm, word_hbm, pos_hbm, out_hbm, idx_v, buf_v, posblk_v, sem):
    wid = lax.axis_index("s") * 2 + lax.axis_index("c")
    base = wid * (ROWS_PER_W * L)

    pltpu.sync_copy(pos_hbm.at[pl.ds(1, L)], posblk_v)

    def chunk_body(c, carry):
        tok0 = base + c * CHUNK
        pltpu.sync_copy(ids_hbm.at[pl.ds(tok0, CHUNK)], idx_v)
        cp0 = pltpu.async_copy(
            word_hbm.at[idx_v.at[pl.ds(0, 128)]], buf_v.at[pl.ds(0, 128)], sem)
        cp1 = pltpu.async_copy(
            word_hbm.at[idx_v.at[pl.ds(128, 72)]], buf_v.at[pl.ds(128, 72)], sem)
        cp0.wait()
        cp1.wait()

        def row_body(r, carry2):
            for j in range(VPR):
                sl = pl.ds(j * LANES, LANES)
                buf_v[r, sl] = buf_v[r, sl] + posblk_v[r, sl]
            return carry2

        lax.fori_loop(0, CHUNK, row_body, 0, unroll=4)
        pltpu.sync_copy(buf_v, out_hbm.at[pl.ds(tok0, CHUNK)])
        return carry

    lax.fori_loop(0, ROWS_PER_W, chunk_body, 0)


@jax.jit
def kernel(input_ids, word_emb, pos_emb):
    ids_flat = input_ids.reshape(-1).astype(jnp.int32)
    mesh = plsc.VectorSubcoreMesh(core_axis_name="c", subcore_axis_name="s")
    out = pl.kernel(
        _body,
        out_type=jax.ShapeDtypeStruct((B * L, HIDDEN), jnp.float32),
        mesh=mesh,
        scratch_types=[
            pltpu.VMEM((CHUNK,), jnp.int32),
            pltpu.VMEM((CHUNK, HIDDEN), jnp.float32),
            pltpu.VMEM((L, HIDDEN), jnp.float32),
            pltpu.SemaphoreType.DMA,
        ],
    )(ids_flat, word_emb, pos_emb)
    return out.reshape(B, L, HIDDEN)


# speedup vs baseline: 1.1506x; 1.1506x over previous
import jax
import jax.numpy as jnp
from jax import lax
from jax.experimental import pallas as pl
from jax.experimental.pallas import tpu as pltpu
from jax.experimental.pallas import tpu_sc as plsc

VOCAB = 1000000
HIDDEN = 64
L = 200
B = 1024
NW = 32
ROWS_PER_W = B // NW
CHUNK = L
LANES = 16
VPR = HIDDEN // LANES


def _body(ids_hbm, word_hbm, pos_hbm, out_hbm, idx_v, buf_v, posblk_v, sem):
    wid = lax.axis_index("s") * 2 + lax.axis_index("c")
    base = wid * (ROWS_PER_W * L)

    pltpu.sync_copy(pos_hbm.at[pl.ds(0, L + 8)], posblk_v)

    def chunk_body(c, carry):
        tok0 = base + c * CHUNK
        pltpu.sync_copy(ids_hbm.at[pl.ds(tok0, CHUNK)], idx_v)
        cp0 = pltpu.async_copy(
            word_hbm.at[idx_v.at[pl.ds(0, 128)]], buf_v.at[pl.ds(0, 128)], sem)
        cp1 = pltpu.async_copy(
            word_hbm.at[idx_v.at[pl.ds(128, 72)]], buf_v.at[pl.ds(128, 72)], sem)
        cp0.wait()
        cp1.wait()

        def row_body(r, carry2):
            for j in range(VPR):
                sl = pl.ds(j * LANES, LANES)
                buf_v[r, sl] = buf_v[r, sl] + posblk_v[r + 1, sl]
            return carry2

        lax.fori_loop(0, CHUNK, row_body, 0, unroll=4)
        pltpu.sync_copy(buf_v, out_hbm.at[pl.ds(tok0, CHUNK)])
        return carry

    lax.fori_loop(0, ROWS_PER_W, chunk_body, 0)


@jax.jit
def kernel(input_ids, word_emb, pos_emb):
    ids_flat = input_ids.reshape(-1).astype(jnp.int32)
    mesh = plsc.VectorSubcoreMesh(core_axis_name="c", subcore_axis_name="s")
    out = pl.kernel(
        _body,
        out_type=jax.ShapeDtypeStruct((B * L, HIDDEN), jnp.float32),
        mesh=mesh,
        compiler_params=pltpu.CompilerParams(use_tc_tiling_on_sc=False),
        scratch_types=[
            pltpu.VMEM((CHUNK,), jnp.int32),
            pltpu.VMEM((CHUNK, HIDDEN), jnp.float32),
            pltpu.VMEM((L + 8, HIDDEN), jnp.float32),
            pltpu.SemaphoreType.DMA,
        ],
    )(ids_flat, word_emb, pos_emb)
    return out.reshape(B, L, HIDDEN)


# speedup vs baseline: 1.3629x; 1.1845x over previous
import jax
import jax.numpy as jnp
from jax import lax
from jax.experimental import pallas as pl
from jax.experimental.pallas import tpu as pltpu
from jax.experimental.pallas import tpu_sc as plsc

VOCAB = 1000000
HIDDEN = 64
L = 200
B = 1024
NW = 32
ROWS_PER_W = B // NW
TOK_PER_W = ROWS_PER_W * L
NCHUNK = ROWS_PER_W
LANES = 16
VPR = HIDDEN // LANES


def _body(ids_hbm, word_hbm, pos_hbm, out_hbm,
          idx_all, posblk, g0, g1, o0, o1, semg0, semg1, sems0, sems1):
    wid = lax.axis_index("s") * 2 + lax.axis_index("c")
    base = wid * TOK_PER_W

    gbuf = (g0, g1)
    obuf = (o0, o1)
    semg = (semg0, semg1)
    sems = (sems0, sems1)

    pltpu.sync_copy(ids_hbm.at[pl.ds(base, TOK_PER_W)], idx_all)
    pltpu.sync_copy(pos_hbm.at[pl.ds(0, L + 8)], posblk)

    def start_gather(c, b):
        off = c * L
        pltpu.async_copy(word_hbm.at[idx_all.at[pl.ds(off, 128)]],
                         gbuf[b].at[pl.ds(0, 128)], semg[b])
        pltpu.async_copy(word_hbm.at[idx_all.at[pl.ds(off + 128, 72)]],
                         gbuf[b].at[pl.ds(128, 72)], semg[b])

    def wait_gather(b):
        pltpu.make_async_copy(word_hbm.at[pl.ds(0, L)], gbuf[b], semg[b]).wait()

    def wait_store(b):
        pltpu.make_async_copy(obuf[b], out_hbm.at[pl.ds(0, L)], sems[b]).wait()

    start_gather(0, 0)

    def group(g, carry):
        for b in range(2):
            c = g * 2 + b

            @pl.when(c < NCHUNK - 1)
            def _():
                start_gather(c + 1, 1 - b)

            wait_gather(b)

            @pl.when(c >= 2)
            def _():
                wait_store(b)

            ob = obuf[b]
            gb = gbuf[b]

            @plsc.parallel_loop(0, L, step=1, unroll=8)
            def _(r):
                for j in range(VPR):
                    sl = pl.ds(j * LANES, LANES)
                    ob[r, sl] = gb[r, sl] + posblk[r + 1, sl]

            pltpu.async_copy(ob, out_hbm.at[pl.ds(base + c * L, L)], sems[b])
        return carry

    lax.fori_loop(0, NCHUNK // 2, group, 0)
    wait_store(0)
    wait_store(1)


@jax.jit
def kernel(input_ids, word_emb, pos_emb):
    ids_flat = input_ids.reshape(-1).astype(jnp.int32)
    mesh = plsc.VectorSubcoreMesh(core_axis_name="c", subcore_axis_name="s")
    out = pl.kernel(
        _body,
        out_type=jax.ShapeDtypeStruct((B * L, HIDDEN), jnp.float32),
        mesh=mesh,
        compiler_params=pltpu.CompilerParams(use_tc_tiling_on_sc=False),
        scratch_types=[
            pltpu.VMEM((TOK_PER_W,), jnp.int32),
            pltpu.VMEM((L + 8, HIDDEN), jnp.float32),
            pltpu.VMEM((L, HIDDEN), jnp.float32),
            pltpu.VMEM((L, HIDDEN), jnp.float32),
            pltpu.VMEM((L, HIDDEN), jnp.float32),
            pltpu.VMEM((L, HIDDEN), jnp.float32),
            pltpu.SemaphoreType.DMA,
            pltpu.SemaphoreType.DMA,
            pltpu.SemaphoreType.DMA,
            pltpu.SemaphoreType.DMA,
        ],
    )(ids_flat, word_emb, pos_emb)
    return out.reshape(B, L, HIDDEN)


# speedup vs baseline: 1.3708x; 1.0058x over previous
import jax
import jax.numpy as jnp
from jax import lax
from jax.experimental import pallas as pl
from jax.experimental.pallas import tpu as pltpu
from jax.experimental.pallas import tpu_sc as plsc

VOCAB = 1000000
HIDDEN = 64
L = 200
B = 1024
NW = 32
ROWS_PER_W = B // NW
TOK_PER_W = ROWS_PER_W * L
NCHUNK = ROWS_PER_W
LANES = 16
VPR = HIDDEN // LANES
GDEPTH = 4
ODEPTH = 2


def _body(ids_hbm, word_hbm, pos_hbm, out_hbm, idx_all, posblk, *rest):
    gbuf = rest[:GDEPTH]
    obuf = rest[GDEPTH:GDEPTH + ODEPTH]
    semg = rest[GDEPTH + ODEPTH:2 * GDEPTH + ODEPTH]
    sems = rest[2 * GDEPTH + ODEPTH:]

    wid = lax.axis_index("s") * 2 + lax.axis_index("c")
    base = wid * TOK_PER_W

    pltpu.sync_copy(ids_hbm.at[pl.ds(base, TOK_PER_W)], idx_all)
    pltpu.sync_copy(pos_hbm.at[pl.ds(0, L + 8)], posblk)

    def start_gather(c, s):
        off = c * L
        pltpu.async_copy(word_hbm.at[idx_all.at[pl.ds(off, 128)]],
                         gbuf[s].at[pl.ds(0, 128)], semg[s])
        pltpu.async_copy(word_hbm.at[idx_all.at[pl.ds(off + 128, 72)]],
                         gbuf[s].at[pl.ds(128, 72)], semg[s])

    def wait_gather(s):
        pltpu.make_async_copy(word_hbm.at[pl.ds(0, L)], gbuf[s], semg[s]).wait()

    def wait_store(s):
        pltpu.make_async_copy(obuf[s], out_hbm.at[pl.ds(0, L)], sems[s]).wait()

    for c in range(GDEPTH - 1):
        start_gather(c, c)

    def group(g, carry):
        for b in range(GDEPTH):
            c = g * GDEPTH + b
            so = b % ODEPTH

            @pl.when(c + GDEPTH - 1 < NCHUNK)
            def _():
                start_gather(c + GDEPTH - 1, (b + GDEPTH - 1) % GDEPTH)

            wait_gather(b)

            @pl.when(c >= ODEPTH)
            def _():
                wait_store(so)

            ob = obuf[so]
            gb = gbuf[b]

            @plsc.parallel_loop(0, L, step=1, unroll=8)
            def _(r):
                for j in range(VPR):
                    sl = pl.ds(j * LANES, LANES)
                    ob[r, sl] = gb[r, sl] + posblk[r + 1, sl]

            pltpu.async_copy(ob, out_hbm.at[pl.ds(base + c * L, L)], sems[so])
        return carry

    lax.fori_loop(0, NCHUNK // GDEPTH, group, 0)
    for s in range(ODEPTH):
        wait_store(s)


@jax.jit
def kernel(input_ids, word_emb, pos_emb):
    ids_flat = input_ids.reshape(-1).astype(jnp.int32)
    mesh = plsc.VectorSubcoreMesh(core_axis_name="c", subcore_axis_name="s")
    out = pl.kernel(
        _body,
        out_type=jax.ShapeDtypeStruct((B * L, HIDDEN), jnp.float32),
        mesh=mesh,
        compiler_params=pltpu.CompilerParams(use_tc_tiling_on_sc=False),
        scratch_types=(
            [pltpu.VMEM((TOK_PER_W,), jnp.int32),
             pltpu.VMEM((L + 8, HIDDEN), jnp.float32)]
            + [pltpu.VMEM((L, HIDDEN), jnp.float32)] * (GDEPTH + ODEPTH)
            + [pltpu.SemaphoreType.DMA] * (GDEPTH + ODEPTH)
        ),
    )(ids_flat, word_emb, pos_emb)
    return out.reshape(B, L, HIDDEN)


# speedup vs baseline: 1.5834x; 1.1551x over previous
import jax
import jax.numpy as jnp
from jax import lax
from jax.experimental import pallas as pl
from jax.experimental.pallas import tpu as pltpu
from jax.experimental.pallas import tpu_sc as plsc

VOCAB = 1000000
HIDDEN = 64
HPAD = 128
L = 200
B = 1024
NW = 32
ROWS_PER_W = B // NW
TOK_PER_W = ROWS_PER_W * L
NCHUNK = ROWS_PER_W
LANES = 16
VPR = HIDDEN // LANES
GDEPTH = 2
ODEPTH = 2


def _body(ids_hbm, word_hbm, pos_hbm, out_hbm, idx_all, posblk, *rest):
    gbuf = rest[:GDEPTH]
    obuf = rest[GDEPTH:GDEPTH + ODEPTH]
    semg = rest[GDEPTH + ODEPTH:2 * GDEPTH + ODEPTH]
    sems = rest[2 * GDEPTH + ODEPTH:]

    wid = lax.axis_index("s") * 2 + lax.axis_index("c")
    base = wid * TOK_PER_W

    pltpu.sync_copy(ids_hbm.at[pl.ds(base, TOK_PER_W)], idx_all)
    pltpu.sync_copy(pos_hbm, posblk)

    def start_gather(c, s):
        off = c * L
        pltpu.async_copy(word_hbm.at[idx_all.at[pl.ds(off, 128)]],
                         gbuf[s].at[pl.ds(0, 128)], semg[s])
        pltpu.async_copy(word_hbm.at[idx_all.at[pl.ds(off + 128, 72)]],
                         gbuf[s].at[pl.ds(128, 72)], semg[s])

    def wait_gather(s):
        pltpu.make_async_copy(word_hbm.at[pl.ds(0, L)], gbuf[s], semg[s]).wait()

    def wait_store(s):
        pltpu.make_async_copy(obuf[s], out_hbm.at[pl.ds(0, L)], sems[s]).wait()

    for c in range(GDEPTH - 1):
        start_gather(c, c)

    def group(g, carry):
        for b in range(GDEPTH):
            c = g * GDEPTH + b
            so = b % ODEPTH

            @pl.when(c + GDEPTH - 1 < NCHUNK)
            def _():
                start_gather(c + GDEPTH - 1, (b + GDEPTH - 1) % GDEPTH)

            wait_gather(b)

            @pl.when(c >= ODEPTH)
            def _():
                wait_store(so)

            ob = obuf[so]
            gb = gbuf[b]

            @plsc.parallel_loop(0, L, step=1, unroll=8)
            def _(r):
                for j in range(VPR):
                    sl = pl.ds(j * LANES, LANES)
                    ob[r, sl] = gb[r, sl] + posblk[pl.ds(r * HIDDEN + j * LANES, LANES)]

            pltpu.async_copy(ob, out_hbm.at[pl.ds(base + c * L, L)], sems[so])
        return carry

    lax.fori_loop(0, NCHUNK // GDEPTH, group, 0)
    for s in range(ODEPTH):
        wait_store(s)


@jax.jit
def kernel(input_ids, word_emb, pos_emb):
    ids_flat = input_ids.reshape(-1).astype(jnp.int32)
    word_pad = jnp.pad(word_emb, ((0, 0), (0, HPAD - HIDDEN)))
    pos_lin = pos_emb[1:L + 1].reshape(-1)
    mesh = plsc.VectorSubcoreMesh(core_axis_name="c", subcore_axis_name="s")
    out = pl.kernel(
        _body,
        out_type=jax.ShapeDtypeStruct((B * L, HPAD), jnp.float32),
        mesh=mesh,
        scratch_types=(
            [pltpu.VMEM((TOK_PER_W,), jnp.int32),
             pltpu.VMEM((L * HIDDEN,), jnp.float32)]
            + [pltpu.VMEM((L, HPAD), jnp.float32)] * (GDEPTH + ODEPTH)
            + [pltpu.SemaphoreType.DMA] * (GDEPTH + ODEPTH)
        ),
    )(ids_flat, word_pad, pos_lin)
    return out.reshape(B, L, HPAD)[:, :, :HIDDEN]
